# Initial kernel scaffold; baseline (speedup 1.0000x reference)
import jax
import jax.numpy as jnp
from jax.experimental import pallas as pl
from jax.experimental.pallas import tpu as pltpu
from jax.experimental.pallas import tpu_sc as plsc

DIM = 2048
GW = 16
TM = 1024
TN = 1280


def _gather_rows(table, ids_flat):
    ntok = ids_flat.shape[1]
    mesh = plsc.VectorSubcoreMesh(core_axis_name="c", subcore_axis_name="s")

    @pl.kernel(
        out_type=jax.ShapeDtypeStruct((ntok, DIM), table.dtype),
        mesh=mesh,
    )
    def gather_kernel(table_hbm, ids_hbm, h_hbm):
        def body(i_vmem, o_vmem):
            pltpu.sync_copy(table_hbm.at[i_vmem.at[0]], o_vmem)

        pltpu.emit_pipeline(
            body,
            grid=(ntok // GW,),
            in_specs=[pl.BlockSpec((1, GW), index_map=lambda i: (0, i))],
            out_specs=[pl.BlockSpec((GW, DIM), index_map=lambda i: (i, 0))],
            core_axis_name=("c", "s"),
            dimension_semantics=(pltpu.PARALLEL,),
        )(ids_hbm, h_hbm)

    return gather_kernel(table, ids_flat)


def _project(h, W, b2d):
    ntok, vocab = h.shape[0], W.shape[0]

    def mm_kernel(h_ref, w_ref, b_ref, o_ref):
        o_ref[...] = jax.lax.dot_general(
            h_ref[...], w_ref[...],
            (((1,), (1,)), ((), ())),
            preferred_element_type=jnp.float32,
        ) + b_ref[...]

    return pl.pallas_call(
        mm_kernel,
        grid=(vocab // TN, ntok // TM),
        in_specs=[
            pl.BlockSpec((TM, DIM), lambda i, j: (j, 0)),
            pl.BlockSpec((TN, DIM), lambda i, j: (i, 0)),
            pl.BlockSpec((1, TN), lambda i, j: (0, i)),
        ],
        out_specs=pl.BlockSpec((TM, TN), lambda i, j: (j, i)),
        out_shape=jax.ShapeDtypeStruct((ntok, vocab), jnp.float32),
    )(h, W, b2d)


def kernel(ids, emb_table, W, b):
    batch, seq = ids.shape
    ids_flat = ids.reshape(1, batch * seq).astype(jnp.int32)
    h = _gather_rows(emb_table, ids_flat)
    logits = _project(h, W, b.reshape(1, -1))
    return logits.reshape(batch, seq, W.shape[0])


# speedup vs baseline: 1.2128x; 1.2128x over previous
import functools

import jax
import jax.numpy as jnp
from jax import lax
from jax.experimental import pallas as pl
from jax.experimental.pallas import tpu as pltpu
from jax.experimental.pallas import tpu_sc as plsc

DIM = 2048
NC = 2
NS = 16
CH = 16
TM = 1024
TN = 1280


def _gather_rows(table, ids_flat):
    ntok = ids_flat.shape[0]
    n_work = NC * NS
    b_per_w = ntok // n_work
    mesh = plsc.VectorSubcoreMesh(core_axis_name="c", subcore_axis_name="s")

    @functools.partial(
        pl.kernel,
        mesh=mesh,
        out_type=jax.ShapeDtypeStruct((ntok, DIM), table.dtype),
        scratch_types=[
            pltpu.VMEM((b_per_w,), jnp.int32),
            pltpu.VMEM((CH, DIM), table.dtype),
            pltpu.SemaphoreType.DMA,
        ],
    )
    def gather_kernel(table_hbm, idx_hbm, out_hbm, idx_v, rows_v, sem):
        wid = lax.axis_index("s") * NC + lax.axis_index("c")
        base = wid * b_per_w
        pltpu.sync_copy(idx_hbm.at[pl.ds(base, b_per_w)], idx_v)

        @pl.loop(0, b_per_w // CH)
        def _(j):
            off = j * CH
            pltpu.async_copy(
                table_hbm.at[idx_v.at[pl.ds(off, CH)]], rows_v, sem
            ).wait()
            pltpu.sync_copy(rows_v, out_hbm.at[pl.ds(base + off, CH)])

    return gather_kernel(table, ids_flat)


def _project(h, W, b2d):
    ntok, vocab = h.shape[0], W.shape[0]

    def mm_kernel(h_ref, w_ref, b_ref, o_ref):
        o_ref[...] = jax.lax.dot_general(
            h_ref[...], w_ref[...],
            (((1,), (1,)), ((), ())),
            preferred_element_type=jnp.float32,
        ) + b_ref[...]

    return pl.pallas_call(
        mm_kernel,
        grid=(vocab // TN, ntok // TM),
        in_specs=[
            pl.BlockSpec((TM, DIM), lambda i, j: (j, 0)),
            pl.BlockSpec((TN, DIM), lambda i, j: (i, 0)),
            pl.BlockSpec((1, TN), lambda i, j: (0, i)),
        ],
        out_specs=pl.BlockSpec((TM, TN), lambda i, j: (j, i)),
        out_shape=jax.ShapeDtypeStruct((ntok, vocab), jnp.float32),
    )(h, W, b2d)


def kernel(ids, emb_table, W, b):
    batch, seq = ids.shape
    ids_flat = ids.reshape(batch * seq).astype(jnp.int32)
    h = _gather_rows(emb_table, ids_flat)
    logits = _project(h, W, b.reshape(1, -1))
    return logits.reshape(batch, seq, W.shape[0])


# speedup vs baseline: 1.3646x; 1.1252x over previous
import functools

import jax
import jax.numpy as jnp
from jax import lax
from jax.experimental import pallas as pl
from jax.experimental.pallas import tpu as pltpu
from jax.experimental.pallas import tpu_sc as plsc

DIM = 2048
NC = 2
NS = 16
CH = 16
TN = 256


def _gather_rows(table, ids_flat):
    ntok = ids_flat.shape[0]
    n_work = NC * NS
    b_per_w = ntok // n_work
    mesh = plsc.VectorSubcoreMesh(core_axis_name="c", subcore_axis_name="s")

    @functools.partial(
        pl.kernel,
        mesh=mesh,
        out_type=jax.ShapeDtypeStruct((ntok, DIM), table.dtype),
        scratch_types=[
            pltpu.VMEM((b_per_w,), jnp.int32),
            pltpu.VMEM((CH, DIM), table.dtype),
            pltpu.SemaphoreType.DMA,
        ],
    )
    def gather_kernel(table_hbm, idx_hbm, out_hbm, idx_v, rows_v, sem):
        wid = lax.axis_index("s") * NC + lax.axis_index("c")
        base = wid * b_per_w
        pltpu.sync_copy(idx_hbm.at[pl.ds(base, b_per_w)], idx_v)

        @pl.loop(0, b_per_w // CH)
        def _(j):
            off = j * CH
            pltpu.async_copy(
                table_hbm.at[idx_v.at[pl.ds(off, CH)]], rows_v, sem
            ).wait()
            pltpu.sync_copy(rows_v, out_hbm.at[pl.ds(base + off, CH)])

    return gather_kernel(table, ids_flat)


def _project(h, W, b2d):
    ntok, vocab = h.shape[0], W.shape[0]

    def mm_kernel(h_ref, w_ref, b_ref, o_ref):
        o_ref[...] = jax.lax.dot_general(
            h_ref[...], w_ref[...],
            (((1,), (1,)), ((), ())),
            preferred_element_type=jnp.float32,
        ) + b_ref[...]

    return pl.pallas_call(
        mm_kernel,
        grid=(vocab // TN,),
        in_specs=[
            pl.BlockSpec((ntok, DIM), lambda i: (0, 0)),
            pl.BlockSpec((TN, DIM), lambda i: (i, 0)),
            pl.BlockSpec((1, TN), lambda i: (0, i)),
        ],
        out_specs=pl.BlockSpec((ntok, TN), lambda i: (0, i)),
        out_shape=jax.ShapeDtypeStruct((ntok, vocab), jnp.float32),
    )(h, W, b2d)


def kernel(ids, emb_table, W, b):
    batch, seq = ids.shape
    ids_flat = ids.reshape(batch * seq).astype(jnp.int32)
    h = _gather_rows(emb_table, ids_flat)
    logits = _project(h, W, b.reshape(1, -1))
    return logits.reshape(batch, seq, W.shape[0])


# speedup vs baseline: 1.3673x; 1.0020x over previous
import functools

import jax
import jax.numpy as jnp
from jax import lax
from jax.experimental import pallas as pl
from jax.experimental.pallas import tpu as pltpu
from jax.experimental.pallas import tpu_sc as plsc

DIM = 2048
NC = 2
NS = 16
CH = 16
TN = 256


def _gather_rows(table, ids_flat):
    ntok = ids_flat.shape[0]
    n_work = NC * NS
    b_per_w = ntok // n_work
    mesh = plsc.VectorSubcoreMesh(core_axis_name="c", subcore_axis_name="s")

    @functools.partial(
        pl.kernel,
        mesh=mesh,
        out_type=jax.ShapeDtypeStruct((ntok, DIM), table.dtype),
        scratch_types=[
            pltpu.VMEM((b_per_w,), jnp.int32),
            pltpu.VMEM((CH, DIM), table.dtype),
            pltpu.SemaphoreType.DMA,
        ],
    )
    def gather_kernel(table_hbm, idx_hbm, out_hbm, idx_v, rows_v, sem):
        wid = lax.axis_index("s") * NC + lax.axis_index("c")
        base = wid * b_per_w
        pltpu.sync_copy(idx_hbm.at[pl.ds(base, b_per_w)], idx_v)

        @pl.loop(0, b_per_w // CH)
        def _(j):
            off = j * CH
            pltpu.async_copy(
                table_hbm.at[idx_v.at[pl.ds(off, CH)]], rows_v, sem
            ).wait()
            pltpu.sync_copy(rows_v, out_hbm.at[pl.ds(base + off, CH)])

    return gather_kernel(table, ids_flat)


def _project(h, W, b2d):
    ntok, vocab = h.shape[0], W.shape[0]

    def mm_kernel(h_ref, w_ref, b_ref, o_ref):
        o_ref[...] = jax.lax.dot_general(
            h_ref[...], w_ref[...],
            (((1,), (1,)), ((), ())),
            preferred_element_type=jnp.float32,
        ) + b_ref[...]

    return pl.pallas_call(
        mm_kernel,
        grid=(vocab // TN,),
        in_specs=[
            pl.BlockSpec((ntok, DIM), lambda i: (0, 0)),
            pl.BlockSpec((TN, DIM), lambda i: (i, 0)),
            pl.BlockSpec((1, TN), lambda i: (0, i)),
        ],
        out_specs=pl.BlockSpec((ntok, TN), lambda i: (0, i)),
        out_shape=jax.ShapeDtypeStruct((ntok, vocab), jnp.float32),
        compiler_params=pltpu.CompilerParams(
            dimension_semantics=("parallel",),
        ),
    )(h, W, b2d)


def kernel(ids, emb_table, W, b):
    batch, seq = ids.shape
    ids_flat = ids.reshape(batch * seq).astype(jnp.int32)
    h = _gather_rows(emb_table, ids_flat)
    logits = _project(h, W, b.reshape(1, -1))
    return logits.reshape(batch, seq, W.shape[0])
